# baseline (speedup 1.0000x reference)
import jax
import jax.numpy as jnp
from jax import lax
from jax.experimental import pallas as pl
from jax.experimental.pallas import tpu as pltpu
from jax.experimental.pallas import tpu_sc as plsc
from jax.experimental import layout as _layout

EMBED = 32
BATCH = 16384
NW = 32
PER_W = BATCH // NW
HALF = PER_W // 2
CHUNK = 128
ROW = 128
TBLK = 8192
QB = TBLK // 4


def _mf_body(user_h, item_p_h, item_n_h, users_q, items_q, out_p_h, out_n_h,
             idx_u, idx_p, idx_n, qid_u, qid_p, qid_n,
             rows_u, rows_p, rows_n, out_p_v, out_n_v, sem):
    wid = lax.axis_index("s") * 2 + lax.axis_index("c")
    base = wid * PER_W
    lane = lax.iota(jnp.int32, 16)

    for half in range(2):
        hbase = base + half * HALF

        cps = [
            pltpu.make_async_copy(user_h.at[pl.ds(hbase, HALF)], idx_u, sem),
            pltpu.make_async_copy(item_p_h.at[pl.ds(hbase, HALF)], idx_p, sem),
            pltpu.make_async_copy(item_n_h.at[pl.ds(hbase, HALF)], idx_n, sem),
        ]
        for c in cps:
            c.start()
        for c in cps:
            c.wait()

        for src, dst in ((idx_u, qid_u), (idx_p, qid_p), (idx_n, qid_n)):
            for c in range(HALF // 16):
                sl = pl.ds(c * 16, 16)
                v = src[sl]
                dst[sl] = lax.shift_left(lax.shift_right_logical(v, 13), 11) \
                    | (v & (QB - 1))

        gathers = []
        for j in range(HALF // CHUNK):
            sl = pl.ds(j * CHUNK, CHUNK)
            gathers.append(pltpu.make_async_copy(
                users_q.at[qid_u.at[sl]], rows_u.at[sl], sem))
            gathers.append(pltpu.make_async_copy(
                items_q.at[qid_p.at[sl]], rows_p.at[sl], sem))
            gathers.append(pltpu.make_async_copy(
                items_q.at[qid_n.at[sl]], rows_n.at[sl], sem))
        for g in gathers:
            g.start()
        for g in gathers:
            g.wait()

        def chunk_body(c, carry):
            sl = pl.ds(c * 16, 16)
            row = c * 16 + lane
            ph_u = lax.shift_left(lax.shift_right_logical(idx_u[sl], 11) & 3, 5)
            ph_p = lax.shift_left(lax.shift_right_logical(idx_p[sl], 11) & 3, 5)
            ph_n = lax.shift_left(lax.shift_right_logical(idx_n[sl], 11) & 3, 5)
            acc_p = jnp.zeros((16,), jnp.float32)
            acc_n = jnp.zeros((16,), jnp.float32)
            for d in range(EMBED):
                rot = (lane + d) & (EMBED - 1)
                u = plsc.load_gather(rows_u, [row, ph_u + rot])
                p = plsc.load_gather(rows_p, [row, ph_p + rot])
                n = plsc.load_gather(rows_n, [row, ph_n + rot])
                acc_p = acc_p + u * p
                acc_n = acc_n + u * n
            out_sl = pl.ds(half * HALF + c * 16, 16)
            out_p_v[out_sl] = acc_p
            out_n_v[out_sl] = acc_n
            return carry

        lax.fori_loop(0, HALF // 16, chunk_body, 0)

    pltpu.sync_copy(out_p_v, out_p_h.at[pl.ds(base, PER_W)])
    pltpu.sync_copy(out_n_v, out_n_h.at[pl.ds(base, PER_W)])


@jax.jit
def _mf(user, item_p, item_n, users_q, items_q):
    mesh = plsc.VectorSubcoreMesh(core_axis_name="c", subcore_axis_name="s")
    f = pl.kernel(
        _mf_body,
        mesh=mesh,
        compiler_params=pltpu.CompilerParams(use_tc_tiling_on_sc=False,
                                             needs_layout_passes=False),
        out_type=(
            jax.ShapeDtypeStruct((BATCH,), jnp.float32),
            jax.ShapeDtypeStruct((BATCH,), jnp.float32),
        ),
        scratch_types=[
            pltpu.VMEM((HALF,), jnp.int32),
            pltpu.VMEM((HALF,), jnp.int32),
            pltpu.VMEM((HALF,), jnp.int32),
            pltpu.VMEM((HALF,), jnp.int32),
            pltpu.VMEM((HALF,), jnp.int32),
            pltpu.VMEM((HALF,), jnp.int32),
            pltpu.VMEM((HALF, ROW), jnp.float32),
            pltpu.VMEM((HALF, ROW), jnp.float32),
            pltpu.VMEM((HALF, ROW), jnp.float32),
            pltpu.VMEM((PER_W,), jnp.float32),
            pltpu.VMEM((PER_W,), jnp.float32),
            pltpu.SemaphoreType.DMA,
        ],
    )
    return f(user, item_p, item_n, users_q, items_q)


def _tp_body(u_ref, i_ref, uo_ref, io_ref):
    tu = u_ref[...].T
    ti = i_ref[...].T
    for k in range(4):
        sl = slice(QB * k, QB * (k + 1))
        uo_ref[:, EMBED * k:EMBED * (k + 1)] = tu[sl, :]
        io_ref[:, EMBED * k:EMBED * (k + 1)] = ti[sl, :]


def _relayout(users_t, items_t):
    n = users_t.shape[1]
    nb = pl.cdiv(n, TBLK)
    out = jax.ShapeDtypeStruct((nb * QB, ROW), jnp.float32)
    return pl.pallas_call(
        _tp_body,
        grid=(nb,),
        in_specs=[pl.BlockSpec((EMBED, TBLK), lambda i: (0, i)),
                  pl.BlockSpec((EMBED, TBLK), lambda i: (0, i))],
        out_specs=[pl.BlockSpec((QB, ROW), lambda i: (i, 0)),
                   pl.BlockSpec((QB, ROW), lambda i: (i, 0))],
        out_shape=(out, out),
    )(users_t, items_t)


def _native_view(table):
    lay = _layout.Layout(major_to_minor=(0, 1), tiling=((8, 128),))
    return _layout.with_layout_constraint(table, lay).T


def kernel(user, item_p, item_n, users_table, items_table):
    users_q, items_q = _relayout(_native_view(users_table),
                                 _native_view(items_table))
    return _mf(user.astype(jnp.int32), item_p.astype(jnp.int32),
               item_n.astype(jnp.int32), users_q, items_q)


# speedup vs baseline: 2.2637x; 2.2637x over previous
import jax
import jax.numpy as jnp
from jax import lax
from jax.experimental import pallas as pl
from jax.experimental.pallas import tpu as pltpu
from jax.experimental.pallas import tpu_sc as plsc
from jax.experimental import layout as _layout

EMBED = 32
BATCH = 16384
NW = 32
PER_W = BATCH // NW
HALF = PER_W // 2
CHUNK = 128
ROW = 128
TBLK = 8192
QB = TBLK // 4


def _mf_body(user_h, item_p_h, item_n_h, users_q, items_q, out_p_h, out_n_h,
             idx_u, idx_p, idx_n, qid_u, qid_p, qid_n,
             rows_u, rows_p, rows_n, out_p_v, out_n_v, sem):
    wid = lax.axis_index("s") * 2 + lax.axis_index("c")
    base = wid * PER_W
    lane = lax.iota(jnp.int32, 16)

    for half in range(2):
        hbase = base + half * HALF

        cps = [
            pltpu.make_async_copy(user_h.at[pl.ds(hbase, HALF)], idx_u, sem),
            pltpu.make_async_copy(item_p_h.at[pl.ds(hbase, HALF)], idx_p, sem),
            pltpu.make_async_copy(item_n_h.at[pl.ds(hbase, HALF)], idx_n, sem),
        ]
        for c in cps:
            c.start()
        for c in cps:
            c.wait()

        for src, dst in ((idx_u, qid_u), (idx_p, qid_p), (idx_n, qid_n)):
            for c in range(HALF // 16):
                sl = pl.ds(c * 16, 16)
                v = src[sl]
                dst[sl] = lax.shift_left(lax.shift_right_logical(v, 13), 11) \
                    | (v & (QB - 1))

        gathers = []
        for j in range(HALF // CHUNK):
            sl = pl.ds(j * CHUNK, CHUNK)
            gathers.append(pltpu.make_async_copy(
                users_q.at[qid_u.at[sl]], rows_u.at[sl], sem))
            gathers.append(pltpu.make_async_copy(
                items_q.at[qid_p.at[sl]], rows_p.at[sl], sem))
            gathers.append(pltpu.make_async_copy(
                items_q.at[qid_n.at[sl]], rows_n.at[sl], sem))
        for g in gathers:
            g.start()
        for g in gathers:
            g.wait()

        def chunk_body(c, carry):
            sl = pl.ds(c * 16, 16)
            row = c * 16 + lane
            ph_u = lax.shift_left(lax.shift_right_logical(idx_u[sl], 11) & 3, 5)
            ph_p = lax.shift_left(lax.shift_right_logical(idx_p[sl], 11) & 3, 5)
            ph_n = lax.shift_left(lax.shift_right_logical(idx_n[sl], 11) & 3, 5)
            acc_p = jnp.zeros((16,), jnp.float32)
            acc_n = jnp.zeros((16,), jnp.float32)
            for d in range(EMBED):
                rot = (lane + d) & (EMBED - 1)
                u = plsc.load_gather(rows_u, [row, ph_u + rot])
                p = plsc.load_gather(rows_p, [row, ph_p + rot])
                n = plsc.load_gather(rows_n, [row, ph_n + rot])
                acc_p = acc_p + u * p
                acc_n = acc_n + u * n
            out_sl = pl.ds(half * HALF + c * 16, 16)
            out_p_v[out_sl] = acc_p
            out_n_v[out_sl] = acc_n
            return carry

        lax.fori_loop(0, HALF // 16, chunk_body, 0)

    pltpu.sync_copy(out_p_v, out_p_h.at[pl.ds(base, PER_W)])
    pltpu.sync_copy(out_n_v, out_n_h.at[pl.ds(base, PER_W)])


@jax.jit
def _mf(user, item_p, item_n, users_q, items_q):
    mesh = plsc.VectorSubcoreMesh(core_axis_name="c", subcore_axis_name="s")
    f = pl.kernel(
        _mf_body,
        mesh=mesh,
        compiler_params=pltpu.CompilerParams(use_tc_tiling_on_sc=False,
                                             needs_layout_passes=False),
        out_type=(
            jax.ShapeDtypeStruct((BATCH,), jnp.float32),
            jax.ShapeDtypeStruct((BATCH,), jnp.float32),
        ),
        scratch_types=[
            pltpu.VMEM((HALF,), jnp.int32),
            pltpu.VMEM((HALF,), jnp.int32),
            pltpu.VMEM((HALF,), jnp.int32),
            pltpu.VMEM((HALF,), jnp.int32),
            pltpu.VMEM((HALF,), jnp.int32),
            pltpu.VMEM((HALF,), jnp.int32),
            pltpu.VMEM((HALF, ROW), jnp.float32),
            pltpu.VMEM((HALF, ROW), jnp.float32),
            pltpu.VMEM((HALF, ROW), jnp.float32),
            pltpu.VMEM((PER_W,), jnp.float32),
            pltpu.VMEM((PER_W,), jnp.float32),
            pltpu.SemaphoreType.DMA,
        ],
    )
    return f(user, item_p, item_n, users_q, items_q)


def _tp_body(u_ref, i_ref, uo_ref, io_ref):
    u = u_ref[...]
    i = i_ref[...]
    uo_ref[...] = jnp.concatenate(
        [u[:, QB * k:QB * (k + 1)] for k in range(4)], axis=0).T
    io_ref[...] = jnp.concatenate(
        [i[:, QB * k:QB * (k + 1)] for k in range(4)], axis=0).T


def _relayout(users_t, items_t):
    n = users_t.shape[1]
    nb = pl.cdiv(n, TBLK)
    out = jax.ShapeDtypeStruct((nb * QB, ROW), jnp.float32)
    return pl.pallas_call(
        _tp_body,
        grid=(nb,),
        in_specs=[pl.BlockSpec((EMBED, TBLK), lambda i: (0, i)),
                  pl.BlockSpec((EMBED, TBLK), lambda i: (0, i))],
        out_specs=[pl.BlockSpec((QB, ROW), lambda i: (i, 0)),
                   pl.BlockSpec((QB, ROW), lambda i: (i, 0))],
        out_shape=(out, out),
    )(users_t, items_t)


def _native_view(table):
    lay = _layout.Layout(major_to_minor=(0, 1), tiling=((8, 128),))
    return _layout.with_layout_constraint(table, lay).T


def kernel(user, item_p, item_n, users_table, items_table):
    users_q, items_q = _relayout(_native_view(users_table),
                                 _native_view(items_table))
    return _mf(user.astype(jnp.int32), item_p.astype(jnp.int32),
               item_n.astype(jnp.int32), users_q, items_q)


# speedup vs baseline: 2.6773x; 1.1827x over previous
import jax
import jax.numpy as jnp
from jax import lax
from jax.experimental import pallas as pl
from jax.experimental.pallas import tpu as pltpu
from jax.experimental.pallas import tpu_sc as plsc
from jax.experimental import layout as _layout

EMBED = 32
BATCH = 16384
NW = 32
PER_W = BATCH // NW
HALF = PER_W // 2
CHUNK = 128
ROW = 128
TBLK = 32768
QB = TBLK // 4
SH_T = TBLK.bit_length() - 1
SH_Q = QB.bit_length() - 1


def _mf_body(user_h, item_p_h, item_n_h, users_q, items_q, out_p_h, out_n_h,
             idx_u, idx_p, idx_n, qid_u, qid_p, qid_n,
             rows_u, rows_p, rows_n, out_p_v, out_n_v, sem):
    wid = lax.axis_index("s") * 2 + lax.axis_index("c")
    base = wid * PER_W
    lane = lax.iota(jnp.int32, 16)

    for half in range(2):
        hbase = base + half * HALF

        cps = [
            pltpu.make_async_copy(user_h.at[pl.ds(hbase, HALF)], idx_u, sem),
            pltpu.make_async_copy(item_p_h.at[pl.ds(hbase, HALF)], idx_p, sem),
            pltpu.make_async_copy(item_n_h.at[pl.ds(hbase, HALF)], idx_n, sem),
        ]
        for c in cps:
            c.start()
        for c in cps:
            c.wait()

        for src, dst in ((idx_u, qid_u), (idx_p, qid_p), (idx_n, qid_n)):
            for c in range(HALF // 16):
                sl = pl.ds(c * 16, 16)
                v = src[sl]
                dst[sl] = lax.shift_left(lax.shift_right_logical(v, SH_T), SH_Q) \
                    | (v & (QB - 1))

        gathers = []
        for j in range(HALF // CHUNK):
            sl = pl.ds(j * CHUNK, CHUNK)
            gathers.append(pltpu.make_async_copy(
                users_q.at[qid_u.at[sl]], rows_u.at[sl], sem))
            gathers.append(pltpu.make_async_copy(
                items_q.at[qid_p.at[sl]], rows_p.at[sl], sem))
            gathers.append(pltpu.make_async_copy(
                items_q.at[qid_n.at[sl]], rows_n.at[sl], sem))
        for g in gathers:
            g.start()
        for g in gathers:
            g.wait()

        def chunk_body(c, carry):
            sl = pl.ds(c * 16, 16)
            row = c * 16 + lane
            ph_u = lax.shift_left(lax.shift_right_logical(idx_u[sl], SH_Q) & 3, 5)
            ph_p = lax.shift_left(lax.shift_right_logical(idx_p[sl], SH_Q) & 3, 5)
            ph_n = lax.shift_left(lax.shift_right_logical(idx_n[sl], SH_Q) & 3, 5)
            acc_p = jnp.zeros((16,), jnp.float32)
            acc_n = jnp.zeros((16,), jnp.float32)
            for d in range(EMBED):
                rot = (lane + d) & (EMBED - 1)
                u = plsc.load_gather(rows_u, [row, ph_u + rot])
                p = plsc.load_gather(rows_p, [row, ph_p + rot])
                n = plsc.load_gather(rows_n, [row, ph_n + rot])
                acc_p = acc_p + u * p
                acc_n = acc_n + u * n
            out_sl = pl.ds(half * HALF + c * 16, 16)
            out_p_v[out_sl] = acc_p
            out_n_v[out_sl] = acc_n
            return carry

        lax.fori_loop(0, HALF // 16, chunk_body, 0)

    pltpu.sync_copy(out_p_v, out_p_h.at[pl.ds(base, PER_W)])
    pltpu.sync_copy(out_n_v, out_n_h.at[pl.ds(base, PER_W)])


@jax.jit
def _mf(user, item_p, item_n, users_q, items_q):
    mesh = plsc.VectorSubcoreMesh(core_axis_name="c", subcore_axis_name="s")
    f = pl.kernel(
        _mf_body,
        mesh=mesh,
        compiler_params=pltpu.CompilerParams(use_tc_tiling_on_sc=False,
                                             needs_layout_passes=False),
        out_type=(
            jax.ShapeDtypeStruct((BATCH,), jnp.float32),
            jax.ShapeDtypeStruct((BATCH,), jnp.float32),
        ),
        scratch_types=[
            pltpu.VMEM((HALF,), jnp.int32),
            pltpu.VMEM((HALF,), jnp.int32),
            pltpu.VMEM((HALF,), jnp.int32),
            pltpu.VMEM((HALF,), jnp.int32),
            pltpu.VMEM((HALF,), jnp.int32),
            pltpu.VMEM((HALF,), jnp.int32),
            pltpu.VMEM((HALF, ROW), jnp.float32),
            pltpu.VMEM((HALF, ROW), jnp.float32),
            pltpu.VMEM((HALF, ROW), jnp.float32),
            pltpu.VMEM((PER_W,), jnp.float32),
            pltpu.VMEM((PER_W,), jnp.float32),
            pltpu.SemaphoreType.DMA,
        ],
    )
    return f(user, item_p, item_n, users_q, items_q)


def _tp_body(u_ref, i_ref, uo_ref, io_ref):
    u = u_ref[...]
    i = i_ref[...]
    uo_ref[...] = jnp.concatenate(
        [u[:, QB * k:QB * (k + 1)] for k in range(4)], axis=0).T
    io_ref[...] = jnp.concatenate(
        [i[:, QB * k:QB * (k + 1)] for k in range(4)], axis=0).T


def _relayout(users_t, items_t):
    n = users_t.shape[1]
    nb = pl.cdiv(n, TBLK)
    out = jax.ShapeDtypeStruct((nb * QB, ROW), jnp.float32)
    return pl.pallas_call(
        _tp_body,
        grid=(nb,),
        in_specs=[pl.BlockSpec((EMBED, TBLK), lambda i: (0, i)),
                  pl.BlockSpec((EMBED, TBLK), lambda i: (0, i))],
        out_specs=[pl.BlockSpec((QB, ROW), lambda i: (i, 0)),
                   pl.BlockSpec((QB, ROW), lambda i: (i, 0))],
        out_shape=(out, out),
    )(users_t, items_t)


def _native_view(table):
    lay = _layout.Layout(major_to_minor=(0, 1), tiling=((8, 128),))
    return _layout.with_layout_constraint(table, lay).T


def kernel(user, item_p, item_n, users_table, items_table):
    users_q, items_q = _relayout(_native_view(users_table),
                                 _native_view(items_table))
    return _mf(user.astype(jnp.int32), item_p.astype(jnp.int32),
               item_n.astype(jnp.int32), users_q, items_q)
